# Initial kernel scaffold; baseline (speedup 1.0000x reference)
import jax
import jax.numpy as jnp
from jax.experimental import pallas as pl


def kernel(nfeats, efeats, edge_index, params):
    raise NotImplementedError("write your pallas kernel here")


# speedup vs baseline: 1.1028x; 1.1028x over previous
import functools

import jax
import jax.numpy as jnp
from jax import lax
from jax.experimental import pallas as pl

N = 10000
E = 160000
ND = 256
ED = 16
H = 256
NE = 8

TBN = 1000
TBE = 2000
NEG = -1e30


def _top2_weights(scores):
    iota = lax.broadcasted_iota(jnp.int32, scores.shape, 1)
    m1 = jnp.max(scores, axis=1, keepdims=True)
    i1 = jnp.min(jnp.where(scores == m1, iota, 128), axis=1, keepdims=True)
    oh1 = iota == i1
    s2 = jnp.where(oh1, NEG, scores)
    m2 = jnp.max(s2, axis=1, keepdims=True)
    i2 = jnp.min(jnp.where(s2 == m2, iota, 128), axis=1, keepdims=True)
    mask = oh1 | (iota == i2)
    e = jnp.where(mask, jnp.exp(scores - m1), 0.0)
    return e / jnp.sum(e, axis=1, keepdims=True)


def _mof_node_body(x_ref, wg_ref, bg_ref, we_ref, be_ref, outl_ref, outr_ref):
    x = x_ref[...]
    scores = jnp.dot(x, wg_ref[...], preferred_element_type=jnp.float32) + bg_ref[...]
    w = _top2_weights(scores)
    eo = jnp.dot(x, we_ref[...], preferred_element_type=jnp.float32) + be_ref[...]
    acc = jnp.zeros((x.shape[0], ND), jnp.float32)
    for ei in range(NE):
        acc = acc + w[:, ei:ei + 1] * eo[:, ei * ND:(ei + 1) * ND]
    outl_ref[...] = acc[:, :128]
    outr_ref[...] = acc[:, 128:]


def _mof_node(x, wg_pad, bg_pad, we_all, be_all):
    grid = (N // TBN,)
    return pl.pallas_call(
        _mof_node_body,
        grid=grid,
        in_specs=[
            pl.BlockSpec((TBN, ND), lambda i: (i, 0)),
            pl.BlockSpec((ND, 128), lambda i: (0, 0)),
            pl.BlockSpec((1, 128), lambda i: (0, 0)),
            pl.BlockSpec((ND, NE * ND), lambda i: (0, 0)),
            pl.BlockSpec((1, NE * ND), lambda i: (0, 0)),
        ],
        out_specs=[
            pl.BlockSpec((TBN, 128), lambda i: (i, 0)),
            pl.BlockSpec((TBN, 128), lambda i: (i, 0)),
        ],
        out_shape=[
            jax.ShapeDtypeStruct((N, 128), jnp.float32),
            jax.ShapeDtypeStruct((N, 128), jnp.float32),
        ],
    )(x, wg_pad, bg_pad, we_all, be_all)


def _mof_edge_body(x_ref, wg_ref, bg_ref, we_ref, be_ref, out_ref):
    x = x_ref[...]
    scores = jnp.dot(x, wg_ref[...], preferred_element_type=jnp.float32) + bg_ref[...]
    w = _top2_weights(scores)
    eo = jnp.dot(x, we_ref[...], preferred_element_type=jnp.float32) + be_ref[...]
    acc = jnp.zeros((x.shape[0], ED), jnp.float32)
    for ei in range(NE):
        acc = acc + w[:, ei:ei + 1] * eo[:, ei * ED:(ei + 1) * ED]
    iota = lax.broadcasted_iota(jnp.int32, (x.shape[0], ED), 1)
    ones_col = jnp.where(iota == 0, 1.0, 0.0)
    out_ref[...] = jnp.concatenate([acc, ones_col], axis=1)


def _mof_edge(x, wg_pad, bg_pad, we_all, be_all):
    grid = (E // TBE,)
    return pl.pallas_call(
        _mof_edge_body,
        grid=grid,
        in_specs=[
            pl.BlockSpec((TBE, ED), lambda i: (i, 0)),
            pl.BlockSpec((ED, 128), lambda i: (0, 0)),
            pl.BlockSpec((1, 128), lambda i: (0, 0)),
            pl.BlockSpec((ED, NE * ED), lambda i: (0, 0)),
            pl.BlockSpec((1, NE * ED), lambda i: (0, 0)),
        ],
        out_specs=pl.BlockSpec((TBE, 2 * ED), lambda i: (i, 0)),
        out_shape=jax.ShapeDtypeStruct((E, 2 * ED), jnp.float32),
    )(x, wg_pad, bg_pad, we_all, be_all)


def _layer1_body(hnl_ref, hnr_ref, ssnl_ref, ssnr_ref, p0_ref, p1_ref,
                 waal_ref, waar_ref, wabl_ref, wabr_ref, wac_ref, ba_ref,
                 wet_ref, web_ref, be_ref,
                 hl_ref, hr_ref, al_ref, ar_ref, bl_ref, br_ref, inv_ref):
    ssp = p0_ref[...] + p1_ref[...]
    sse = ssp[:, :ED]
    cnt = ssp[:, ED:ED + 1]
    inv = 1.0 / jnp.maximum(cnt, 1.0)
    dot = functools.partial(jnp.dot, preferred_element_type=jnp.float32)
    u = (dot(ssnl_ref[...], wabl_ref[...]) + dot(ssnr_ref[...], wabr_ref[...])
         + dot(sse, wac_ref[...]))
    h = dot(hnl_ref[...], waal_ref[...]) + dot(hnr_ref[...], waar_ref[...])
    h = jnp.maximum(h + inv * u + ba_ref[...], 0.0)
    a = dot(h, wet_ref[...])
    b = dot(h, web_ref[...]) + be_ref[...]
    hl_ref[...] = h[:, :128]
    hr_ref[...] = h[:, 128:]
    al_ref[...] = a[:, :128]
    ar_ref[...] = a[:, 128:]
    bl_ref[...] = b[:, :128]
    br_ref[...] = b[:, 128:]
    inv_ref[...] = jnp.broadcast_to(inv, (inv.shape[0], 128))


def _layer1(hnl, hnr, ssnl, ssnr, p0, p1, wa, ba, we, be):
    grid = (N // TBN,)
    row = lambda i: (i, 0)
    full = lambda i: (0, 0)
    outs = [jax.ShapeDtypeStruct((N, 128), jnp.float32)] * 7
    return pl.pallas_call(
        _layer1_body,
        grid=grid,
        in_specs=[
            pl.BlockSpec((TBN, 128), row), pl.BlockSpec((TBN, 128), row),
            pl.BlockSpec((TBN, 128), row), pl.BlockSpec((TBN, 128), row),
            pl.BlockSpec((TBN, 2 * ED), row), pl.BlockSpec((TBN, 2 * ED), row),
            pl.BlockSpec((128, H), full), pl.BlockSpec((128, H), full),
            pl.BlockSpec((128, H), full), pl.BlockSpec((128, H), full),
            pl.BlockSpec((ED, H), full), pl.BlockSpec((1, H), full),
            pl.BlockSpec((H, H), full), pl.BlockSpec((H, H), full),
            pl.BlockSpec((1, H), full),
        ],
        out_specs=[pl.BlockSpec((TBN, 128), row)] * 7,
        out_shape=outs,
    )(hnl, hnr, ssnl, ssnr, p0, p1,
      wa[0:128], wa[128:256], wa[256:384], wa[384:512], wa[512:528],
      ba.reshape(1, H), we[:H], we[H:], be.reshape(1, H))


def _layer2_body(hnl_ref, hnr_ref, ssnl_ref, ssnr_ref, ssel_ref, sser_ref,
                 inv_ref,
                 waal_ref, waar_ref, wabl_ref, wabr_ref, wacl_ref, wacr_ref,
                 ba_ref, wet_ref, web_ref, be_ref,
                 hl_ref, hr_ref, al_ref, ar_ref, bl_ref, br_ref):
    inv = inv_ref[:, 0:1]
    dot = functools.partial(jnp.dot, preferred_element_type=jnp.float32)
    u = (dot(ssnl_ref[...], wabl_ref[...]) + dot(ssnr_ref[...], wabr_ref[...])
         + dot(ssel_ref[...], wacl_ref[...]) + dot(sser_ref[...], wacr_ref[...]))
    h = dot(hnl_ref[...], waal_ref[...]) + dot(hnr_ref[...], waar_ref[...])
    h = jnp.maximum(h + inv * u + ba_ref[...], 0.0)
    a = dot(h, wet_ref[...])
    b = dot(h, web_ref[...]) + be_ref[...]
    hl_ref[...] = h[:, :128]
    hr_ref[...] = h[:, 128:]
    al_ref[...] = a[:, :128]
    ar_ref[...] = a[:, 128:]
    bl_ref[...] = b[:, :128]
    br_ref[...] = b[:, 128:]


def _layer2(hnl, hnr, ssnl, ssnr, ssel, sser, inv128, wa, ba, we, be):
    grid = (N // TBN,)
    row = lambda i: (i, 0)
    full = lambda i: (0, 0)
    outs = [jax.ShapeDtypeStruct((N, 128), jnp.float32)] * 6
    return pl.pallas_call(
        _layer2_body,
        grid=grid,
        in_specs=[
            pl.BlockSpec((TBN, 128), row), pl.BlockSpec((TBN, 128), row),
            pl.BlockSpec((TBN, 128), row), pl.BlockSpec((TBN, 128), row),
            pl.BlockSpec((TBN, 128), row), pl.BlockSpec((TBN, 128), row),
            pl.BlockSpec((TBN, 128), row),
            pl.BlockSpec((128, H), full), pl.BlockSpec((128, H), full),
            pl.BlockSpec((128, H), full), pl.BlockSpec((128, H), full),
            pl.BlockSpec((128, H), full), pl.BlockSpec((128, H), full),
            pl.BlockSpec((1, H), full),
            pl.BlockSpec((H, H), full), pl.BlockSpec((H, H), full),
            pl.BlockSpec((1, H), full),
        ],
        out_specs=[pl.BlockSpec((TBN, 128), row)] * 6,
        out_shape=outs,
    )(hnl, hnr, ssnl, ssnr, ssel, sser, inv128,
      wa[0:128], wa[128:256], wa[256:384], wa[384:512], wa[512:640],
      wa[640:768], ba.reshape(1, H), we[:H], we[H:], be.reshape(1, H))


def _segsum_rows(tbl_l, tbl_r, src, dst):
    tbl = jnp.concatenate([tbl_l, tbl_r], axis=1)
    s = jax.ops.segment_sum(tbl[src], dst, num_segments=N)
    return s[:, :128], s[:, 128:]


def _edge_sums(he0aug, dst):
    s = jax.ops.segment_sum(he0aug, dst, num_segments=N)
    return s, jnp.zeros_like(s)


def _fused_edge_scatter(al, ar, bl, br, src, dst):
    a = jnp.concatenate([al, ar], axis=1)
    b = jnp.concatenate([bl, br], axis=1)
    he = jnp.maximum(a[src] + b[dst], 0.0)
    s = jax.ops.segment_sum(he, dst, num_segments=N)
    return s[:, :128], s[:, 128:]


def _edge_out(al, ar, bl, br, src, dst):
    a = jnp.concatenate([al, ar], axis=1)
    b = jnp.concatenate([bl, br], axis=1)
    return jnp.maximum(a[src] + b[dst], 0.0)


def kernel(nfeats, efeats, edge_index, params):
    p = params
    src = edge_index[0].astype(jnp.int32)
    dst = edge_index[1].astype(jnp.int32)

    wg_n = jnp.zeros((ND, 128), jnp.float32).at[:, :NE].set(p['mofn_Wg'])
    bg_n = jnp.full((1, 128), NEG, jnp.float32).at[0, :NE].set(p['mofn_bg'])
    we_n = p['mofn_We'].transpose(1, 0, 2).reshape(ND, NE * ND)
    be_n = p['mofn_be'].reshape(1, NE * ND)
    wg_e = jnp.zeros((ED, 128), jnp.float32).at[:, :NE].set(p['mofe_Wg'])
    bg_e = jnp.full((1, 128), NEG, jnp.float32).at[0, :NE].set(p['mofe_bg'])
    we_e = p['mofe_We'].transpose(1, 0, 2).reshape(ED, NE * ED)
    be_e = p['mofe_be'].reshape(1, NE * ED)

    hn0l, hn0r = _mof_node(nfeats, wg_n, bg_n, we_n, be_n)
    he0aug = _mof_edge(efeats, wg_e, bg_e, we_e, be_e)

    ssn1l, ssn1r = _segsum_rows(hn0l, hn0r, src, dst)
    p0, p1 = _edge_sums(he0aug, dst)
    h1l, h1r, a1l, a1r, b1l, b1r, inv128 = _layer1(
        hn0l, hn0r, ssn1l, ssn1r, p0, p1,
        p['l1_Wa'], p['l1_ba'], p['l1_We'], p['l1_be'])

    ssn2l, ssn2r = _segsum_rows(h1l, h1r, src, dst)
    sse2l, sse2r = _fused_edge_scatter(a1l, a1r, b1l, b1r, src, dst)
    h2l, h2r, a2l, a2r, b2l, b2r = _layer2(
        h1l, h1r, ssn2l, ssn2r, sse2l, sse2r, inv128,
        p['l2_Wa'], p['l2_ba'], p['l2_We'], p['l2_be'])

    hn = jnp.concatenate([h2l, h2r], axis=1)
    he = _edge_out(a2l, a2r, b2l, b2r, src, dst)
    return hn, he


# speedup vs baseline: 2.3582x; 2.1384x over previous
import functools

import jax
import jax.numpy as jnp
from jax import lax
from jax.experimental import pallas as pl
from jax.experimental.pallas import tpu as pltpu
from jax.experimental.pallas import tpu_sc as plsc

N = 10000
E = 160000
ND = 256
ED = 16
H = 256
NE = 8

TBN = 1000
TBE = 2000
NEG = -1e30


def _top2_weights(scores):
    iota = lax.broadcasted_iota(jnp.int32, scores.shape, 1)
    m1 = jnp.max(scores, axis=1, keepdims=True)
    i1 = jnp.min(jnp.where(scores == m1, iota, 128), axis=1, keepdims=True)
    oh1 = iota == i1
    s2 = jnp.where(oh1, NEG, scores)
    m2 = jnp.max(s2, axis=1, keepdims=True)
    i2 = jnp.min(jnp.where(s2 == m2, iota, 128), axis=1, keepdims=True)
    mask = oh1 | (iota == i2)
    e = jnp.where(mask, jnp.exp(scores - m1), 0.0)
    return e / jnp.sum(e, axis=1, keepdims=True)


def _mof_node_body(x_ref, wg_ref, bg_ref, we_ref, be_ref, out_ref):
    x = x_ref[...]
    scores = jnp.dot(x, wg_ref[...], preferred_element_type=jnp.float32) + bg_ref[...]
    w = _top2_weights(scores)
    eo = jnp.dot(x, we_ref[...], preferred_element_type=jnp.float32) + be_ref[...]
    acc = jnp.zeros((x.shape[0], ND), jnp.float32)
    for ei in range(NE):
        acc = acc + w[:, ei:ei + 1] * eo[:, ei * ND:(ei + 1) * ND]
    out_ref[...] = acc


def _mof_node(x, wg_pad, bg_pad, we_all, be_all):
    return pl.pallas_call(
        _mof_node_body,
        grid=(N // TBN,),
        in_specs=[
            pl.BlockSpec((TBN, ND), lambda i: (i, 0)),
            pl.BlockSpec((ND, 128), lambda i: (0, 0)),
            pl.BlockSpec((1, 128), lambda i: (0, 0)),
            pl.BlockSpec((ND, NE * ND), lambda i: (0, 0)),
            pl.BlockSpec((1, NE * ND), lambda i: (0, 0)),
        ],
        out_specs=pl.BlockSpec((TBN, ND), lambda i: (i, 0)),
        out_shape=jax.ShapeDtypeStruct((N, ND), jnp.float32),
    )(x, wg_pad, bg_pad, we_all, be_all)


def _mof_edge_body(x_ref, wg_ref, bg_ref, we_ref, be_ref, out_ref):
    x = x_ref[...]
    scores = jnp.dot(x, wg_ref[...], preferred_element_type=jnp.float32) + bg_ref[...]
    w = _top2_weights(scores)
    eo = jnp.dot(x, we_ref[...], preferred_element_type=jnp.float32) + be_ref[...]
    acc = jnp.zeros((x.shape[0], ED), jnp.float32)
    for ei in range(NE):
        acc = acc + w[:, ei:ei + 1] * eo[:, ei * ED:(ei + 1) * ED]
    iota = lax.broadcasted_iota(jnp.int32, (x.shape[0], ED), 1)
    ones_col = jnp.where(iota == 0, 1.0, 0.0)
    pad = jnp.zeros((x.shape[0], 128 - 2 * ED), jnp.float32)
    out_ref[...] = jnp.concatenate([acc, ones_col, pad], axis=1)


def _mof_edge(x, wg_pad, bg_pad, we_all, be_all):
    return pl.pallas_call(
        _mof_edge_body,
        grid=(E // TBE,),
        in_specs=[
            pl.BlockSpec((TBE, ED), lambda i: (i, 0)),
            pl.BlockSpec((ED, 128), lambda i: (0, 0)),
            pl.BlockSpec((1, 128), lambda i: (0, 0)),
            pl.BlockSpec((ED, NE * ED), lambda i: (0, 0)),
            pl.BlockSpec((1, NE * ED), lambda i: (0, 0)),
        ],
        out_specs=pl.BlockSpec((TBE, 128), lambda i: (i, 0)),
        out_shape=jax.ShapeDtypeStruct((E, 128), jnp.float32),
    )(x, wg_pad, bg_pad, we_all, be_all)


def _layer1_body(hn_ref, ssn_ref, p0_ref, p1_ref,
                 waa_ref, wab_ref, wac_ref, ba_ref,
                 wet_ref, web_ref, be_ref,
                 h_ref, a_ref, b_ref, inv_ref):
    ssp = p0_ref[...] + p1_ref[...]
    sse = ssp[:, :ED]
    cnt = ssp[:, ED:ED + 1]
    inv = 1.0 / jnp.maximum(cnt, 1.0)
    dot = functools.partial(jnp.dot, preferred_element_type=jnp.float32)
    u = dot(ssn_ref[...], wab_ref[...]) + dot(sse, wac_ref[...])
    h = dot(hn_ref[...], waa_ref[...])
    h = jnp.maximum(h + inv * u + ba_ref[...], 0.0)
    h_ref[...] = h
    a_ref[...] = dot(h, wet_ref[...])
    b_ref[...] = dot(h, web_ref[...]) + be_ref[...]
    inv_ref[...] = jnp.broadcast_to(inv, (inv.shape[0], 128))


def _layer1(hn, ssn, p0, p1, wa, ba, we, be):
    row = lambda i: (i, 0)
    full = lambda i: (0, 0)
    return pl.pallas_call(
        _layer1_body,
        grid=(N // TBN,),
        in_specs=[
            pl.BlockSpec((TBN, ND), row), pl.BlockSpec((TBN, ND), row),
            pl.BlockSpec((TBN, 128), row), pl.BlockSpec((TBN, 128), row),
            pl.BlockSpec((ND, H), full), pl.BlockSpec((ND, H), full),
            pl.BlockSpec((ED, H), full), pl.BlockSpec((1, H), full),
            pl.BlockSpec((H, H), full), pl.BlockSpec((H, H), full),
            pl.BlockSpec((1, H), full),
        ],
        out_specs=[pl.BlockSpec((TBN, H), row)] * 3 + [pl.BlockSpec((TBN, 128), row)],
        out_shape=[jax.ShapeDtypeStruct((N, H), jnp.float32)] * 3
        + [jax.ShapeDtypeStruct((N, 128), jnp.float32)],
    )(hn, ssn, p0, p1,
      wa[0:ND], wa[ND:2 * ND], wa[2 * ND:], ba.reshape(1, H),
      we[:H], we[H:], be.reshape(1, H))


def _layer2_body(hn_ref, ssn_ref, sse_ref, inv_ref,
                 waa_ref, wab_ref, wac_ref, ba_ref,
                 wet_ref, web_ref, be_ref,
                 h_ref, a_ref, b_ref):
    inv = inv_ref[:, 0:1]
    dot = functools.partial(jnp.dot, preferred_element_type=jnp.float32)
    u = dot(ssn_ref[...], wab_ref[...]) + dot(sse_ref[...], wac_ref[...])
    h = dot(hn_ref[...], waa_ref[...])
    h = jnp.maximum(h + inv * u + ba_ref[...], 0.0)
    h_ref[...] = h
    a_ref[...] = dot(h, wet_ref[...])
    b_ref[...] = dot(h, web_ref[...]) + be_ref[...]


def _layer2(hn, ssn, sse, inv128, wa, ba, we, be):
    row = lambda i: (i, 0)
    full = lambda i: (0, 0)
    return pl.pallas_call(
        _layer2_body,
        grid=(N // TBN,),
        in_specs=[
            pl.BlockSpec((TBN, ND), row), pl.BlockSpec((TBN, ND), row),
            pl.BlockSpec((TBN, ND), row), pl.BlockSpec((TBN, 128), row),
            pl.BlockSpec((ND, H), full), pl.BlockSpec((ND, H), full),
            pl.BlockSpec((ND, H), full), pl.BlockSpec((1, H), full),
            pl.BlockSpec((H, H), full), pl.BlockSpec((H, H), full),
            pl.BlockSpec((1, H), full),
        ],
        out_specs=[pl.BlockSpec((TBN, H), row)] * 3,
        out_shape=[jax.ShapeDtypeStruct((N, H), jnp.float32)] * 3,
    )(hn, ssn, sse, inv128,
      wa[0:ND], wa[ND:2 * ND], wa[2 * ND:], ba.reshape(1, H),
      we[:H], we[H:], be.reshape(1, H))


@functools.cache
def _sc_mesh():
    return plsc.VectorSubcoreMesh(core_axis_name="c", subcore_axis_name="s")


_NT = 16
NP = 10240
_ZR = NP // _NT
_EPT = E // _NT
_CK = 128
_NCH = _EPT // _CK
_TAIL = _EPT - _NCH * _CK
_EPW = E // (2 * _NT)
_NCHW = _EPW // _CK
_TAILW = _EPW - _NCHW * _CK


def _relu_add_rows(ra, rb, nrows):
    def row(r, carry):
        for j in range(8):
            sl = pl.ds(j * 16, 16)
            va = ra[r, sl]
            vb = rb[r, sl]
            ra[r, sl] = jnp.maximum(va + vb, 0.0)
        return carry
    lax.fori_loop(0, nrows, row, 0)


def _mangle_idx(dst_buf, src_buf, n, scale, off):
    for j in range(n // 16):
        sl = pl.ds(j * 16, 16)
        dst_buf[sl] = src_buf[sl] * scale + off


def _segsum_rows(tbl2, src, dst):
    @functools.partial(
        pl.kernel,
        mesh=_sc_mesh(),
        out_type=jax.ShapeDtypeStruct((2 * NP, 128), jnp.float32),
        scratch_types=[
            pltpu.VMEM_SHARED((NP, 128), jnp.float32),
            pltpu.VMEM((_CK,), jnp.int32), pltpu.VMEM((_CK,), jnp.int32),
            pltpu.VMEM((_TAIL,), jnp.int32), pltpu.VMEM((_TAIL,), jnp.int32),
            pltpu.VMEM((_CK, 128), jnp.float32),
            pltpu.VMEM((_TAIL, 128), jnp.float32),
            pltpu.SemaphoreType.DMA,
        ],
    )
    def k(tbl, src_h, dst_h, zz, out,
          acc, idx_s, idx_d, idx_st, idx_dt, rows, rows_t, sem):
        c = lax.axis_index("c")
        s = lax.axis_index("s")
        pltpu.sync_copy(zz.at[pl.ds(s * _ZR, _ZR)], acc.at[pl.ds(s * _ZR, _ZR)])
        plsc.subcore_barrier()
        base0 = s * _EPT

        def chunk(i, carry):
            base = base0 + i * _CK
            pltpu.sync_copy(src_h.at[pl.ds(base, _CK)], idx_s)
            pltpu.sync_copy(dst_h.at[pl.ds(base, _CK)], idx_d)
            _mangle_idx(idx_s, idx_s, _CK, 2, c)
            pltpu.async_copy(tbl.at[idx_s], rows, sem).wait()
            pltpu.sync_copy(rows, acc.at[idx_d], add=True)
            return carry

        lax.fori_loop(0, _NCH, chunk, 0)
        baset = base0 + _NCH * _CK
        pltpu.sync_copy(src_h.at[pl.ds(baset, _TAIL)], idx_st)
        pltpu.sync_copy(dst_h.at[pl.ds(baset, _TAIL)], idx_dt)
        _mangle_idx(idx_st, idx_st, _TAIL, 2, c)
        pltpu.async_copy(tbl.at[idx_st], rows_t, sem).wait()
        pltpu.sync_copy(rows_t, acc.at[idx_dt], add=True)
        plsc.subcore_barrier()
        pltpu.sync_copy(acc.at[pl.ds(s * _ZR, _ZR)],
                        out.at[pl.ds(c * NP + s * _ZR, _ZR)])

    zz = jnp.zeros((NP, 128), jnp.float32)
    return k(tbl2, src, dst, zz)


def _edge_sums(he0aug, dst):
    @functools.partial(
        pl.kernel,
        mesh=_sc_mesh(),
        out_type=jax.ShapeDtypeStruct((2 * NP, 128), jnp.float32),
        scratch_types=[
            pltpu.VMEM_SHARED((NP, 128), jnp.float32),
            pltpu.VMEM((_CK,), jnp.int32), pltpu.VMEM((_TAILW,), jnp.int32),
            pltpu.VMEM((_CK, 128), jnp.float32),
            pltpu.VMEM((_TAILW, 128), jnp.float32),
        ],
    )
    def k(hea, dst_h, zz, out, acc, idx_d, idx_dt, rows, rows_t):
        c = lax.axis_index("c")
        s = lax.axis_index("s")
        pltpu.sync_copy(zz.at[pl.ds(s * _ZR, _ZR)], acc.at[pl.ds(s * _ZR, _ZR)])
        plsc.subcore_barrier()
        base0 = (s * 2 + c) * _EPW

        def chunk(i, carry):
            base = base0 + i * _CK
            pltpu.sync_copy(hea.at[pl.ds(base, _CK)], rows)
            pltpu.sync_copy(dst_h.at[pl.ds(base, _CK)], idx_d)
            pltpu.sync_copy(rows, acc.at[idx_d], add=True)
            return carry

        lax.fori_loop(0, _NCHW, chunk, 0)
        baset = base0 + _NCHW * _CK
        pltpu.sync_copy(hea.at[pl.ds(baset, _TAILW)], rows_t)
        pltpu.sync_copy(dst_h.at[pl.ds(baset, _TAILW)], idx_dt)
        pltpu.sync_copy(rows_t, acc.at[idx_dt], add=True)
        plsc.subcore_barrier()
        pltpu.sync_copy(acc.at[pl.ds(s * _ZR, _ZR)],
                        out.at[pl.ds(c * NP + s * _ZR, _ZR)])

    zz = jnp.zeros((NP, 128), jnp.float32)
    return k(he0aug, dst, zz)


def _fused_edge_scatter(a2, b2, src, dst):
    @functools.partial(
        pl.kernel,
        mesh=_sc_mesh(),
        out_type=jax.ShapeDtypeStruct((2 * NP, 128), jnp.float32),
        scratch_types=[
            pltpu.VMEM_SHARED((NP, 128), jnp.float32),
            pltpu.VMEM((_CK,), jnp.int32), pltpu.VMEM((_CK,), jnp.int32),
            pltpu.VMEM((_CK,), jnp.int32),
            pltpu.VMEM((_TAIL,), jnp.int32), pltpu.VMEM((_TAIL,), jnp.int32),
            pltpu.VMEM((_TAIL,), jnp.int32),
            pltpu.VMEM((_CK, 128), jnp.float32), pltpu.VMEM((_CK, 128), jnp.float32),
            pltpu.VMEM((_TAIL, 128), jnp.float32), pltpu.VMEM((_TAIL, 128), jnp.float32),
            pltpu.SemaphoreType.DMA,
        ],
    )
    def k(a_t, b_t, src_h, dst_h, zz, out,
          acc, idx_s, idx_d, idx_g, idx_st, idx_dt, idx_gt, ra, rb, rat, rbt, sem):
        c = lax.axis_index("c")
        s = lax.axis_index("s")
        pltpu.sync_copy(zz.at[pl.ds(s * _ZR, _ZR)], acc.at[pl.ds(s * _ZR, _ZR)])
        plsc.subcore_barrier()
        base0 = s * _EPT

        def chunk(i, carry):
            base = base0 + i * _CK
            pltpu.sync_copy(src_h.at[pl.ds(base, _CK)], idx_s)
            pltpu.sync_copy(dst_h.at[pl.ds(base, _CK)], idx_d)
            _mangle_idx(idx_s, idx_s, _CK, 2, c)
            _mangle_idx(idx_g, idx_d, _CK, 2, c)
            pltpu.async_copy(a_t.at[idx_s], ra, sem).wait()
            pltpu.async_copy(b_t.at[idx_g], rb, sem).wait()
            _relu_add_rows(ra, rb, _CK)
            pltpu.sync_copy(ra, acc.at[idx_d], add=True)
            return carry

        lax.fori_loop(0, _NCH, chunk, 0)
        baset = base0 + _NCH * _CK
        pltpu.sync_copy(src_h.at[pl.ds(baset, _TAIL)], idx_st)
        pltpu.sync_copy(dst_h.at[pl.ds(baset, _TAIL)], idx_dt)
        _mangle_idx(idx_st, idx_st, _TAIL, 2, c)
        _mangle_idx(idx_gt, idx_dt, _TAIL, 2, c)
        pltpu.async_copy(a_t.at[idx_st], rat, sem).wait()
        pltpu.async_copy(b_t.at[idx_gt], rbt, sem).wait()
        _relu_add_rows(rat, rbt, _TAIL)
        pltpu.sync_copy(rat, acc.at[idx_dt], add=True)
        plsc.subcore_barrier()
        pltpu.sync_copy(acc.at[pl.ds(s * _ZR, _ZR)],
                        out.at[pl.ds(c * NP + s * _ZR, _ZR)])

    zz = jnp.zeros((NP, 128), jnp.float32)
    return k(a2, b2, src, dst, zz)


def _edge_out(a2, b2, src, dst):
    @functools.partial(
        pl.kernel,
        mesh=_sc_mesh(),
        out_type=jax.ShapeDtypeStruct((2 * E, 128), jnp.float32),
        scratch_types=[
            pltpu.VMEM((_CK,), jnp.int32), pltpu.VMEM((_CK,), jnp.int32),
            pltpu.VMEM((_TAIL,), jnp.int32), pltpu.VMEM((_TAIL,), jnp.int32),
            pltpu.VMEM((_CK, 128), jnp.float32), pltpu.VMEM((_CK, 128), jnp.float32),
            pltpu.VMEM((_TAIL, 128), jnp.float32), pltpu.VMEM((_TAIL, 128), jnp.float32),
            pltpu.SemaphoreType.DMA,
        ],
    )
    def k(a_t, b_t, src_h, dst_h, out,
          idx_s, idx_d, idx_st, idx_dt, ra, rb, rat, rbt, sem):
        c = lax.axis_index("c")
        s = lax.axis_index("s")
        base0 = s * _EPT

        def chunk(i, carry):
            base = base0 + i * _CK
            pltpu.sync_copy(src_h.at[pl.ds(base, _CK)], idx_s)
            pltpu.sync_copy(dst_h.at[pl.ds(base, _CK)], idx_d)
            _mangle_idx(idx_s, idx_s, _CK, 2, c)
            _mangle_idx(idx_d, idx_d, _CK, 2, c)
            pltpu.async_copy(a_t.at[idx_s], ra, sem).wait()
            pltpu.async_copy(b_t.at[idx_d], rb, sem).wait()
            _relu_add_rows(ra, rb, _CK)
            pltpu.sync_copy(ra, out.at[pl.ds(c * E + base, _CK)])
            return carry

        lax.fori_loop(0, _NCH, chunk, 0)
        baset = base0 + _NCH * _CK
        pltpu.sync_copy(src_h.at[pl.ds(baset, _TAIL)], idx_st)
        pltpu.sync_copy(dst_h.at[pl.ds(baset, _TAIL)], idx_dt)
        _mangle_idx(idx_st, idx_st, _TAIL, 2, c)
        _mangle_idx(idx_dt, idx_dt, _TAIL, 2, c)
        pltpu.async_copy(a_t.at[idx_st], rat, sem).wait()
        pltpu.async_copy(b_t.at[idx_dt], rbt, sem).wait()
        _relu_add_rows(rat, rbt, _TAIL)
        pltpu.sync_copy(rat, out.at[pl.ds(c * E + baset, _TAIL)])

    return k(a2, b2, src, dst)


def _unsplit(st):
    return jnp.concatenate([st[:N], st[NP:NP + N]], axis=1)


def kernel(nfeats, efeats, edge_index, params):
    p = params
    src = edge_index[0].astype(jnp.int32)
    dst = edge_index[1].astype(jnp.int32)

    wg_n = jnp.zeros((ND, 128), jnp.float32).at[:, :NE].set(p['mofn_Wg'])
    bg_n = jnp.full((1, 128), NEG, jnp.float32).at[0, :NE].set(p['mofn_bg'])
    we_n = p['mofn_We'].transpose(1, 0, 2).reshape(ND, NE * ND)
    be_n = p['mofn_be'].reshape(1, NE * ND)
    wg_e = jnp.zeros((ED, 128), jnp.float32).at[:, :NE].set(p['mofe_Wg'])
    bg_e = jnp.full((1, 128), NEG, jnp.float32).at[0, :NE].set(p['mofe_bg'])
    we_e = p['mofe_We'].transpose(1, 0, 2).reshape(ED, NE * ED)
    be_e = p['mofe_be'].reshape(1, NE * ED)

    hn0 = _mof_node(nfeats, wg_n, bg_n, we_n, be_n)
    he0aug = _mof_edge(efeats, wg_e, bg_e, we_e, be_e)

    ssn1 = _unsplit(_segsum_rows(hn0.reshape(2 * N, 128), src, dst))
    ps = _edge_sums(he0aug, dst)
    h1, a1, b1, inv128 = _layer1(
        hn0, ssn1, ps[:N], ps[NP:NP + N],
        p['l1_Wa'], p['l1_ba'], p['l1_We'], p['l1_be'])

    ssn2 = _unsplit(_segsum_rows(h1.reshape(2 * N, 128), src, dst))
    sse2 = _unsplit(_fused_edge_scatter(
        a1.reshape(2 * N, 128), b1.reshape(2 * N, 128), src, dst))
    h2, a2, b2 = _layer2(
        h1, ssn2, sse2, inv128,
        p['l2_Wa'], p['l2_ba'], p['l2_We'], p['l2_be'])

    he2 = _edge_out(a2.reshape(2 * N, 128), b2.reshape(2 * N, 128), src, dst)
    he = jnp.concatenate([he2[:E], he2[E:]], axis=1)
    return h2, he
